# baseline (speedup 1.0000x reference)
import functools
from math import sqrt

import jax
import jax.numpy as jnp
from jax import lax
from jax.experimental import pallas as pl
from jax.experimental.pallas import tpu as pltpu
from jax.experimental.pallas import tpu_sc as plsc

D_MODEL = 1024
SCALE = sqrt(D_MODEL)
NBUF = 3


@functools.cache
def _make_sc_lookup(B: int, D: int):
    info = plsc.get_sparse_core_info()
    NC, NS, L = info.num_cores, info.num_subcores, info.num_lanes
    NW = NC * NS
    assert B % NW == 0 and D % L == 0
    b_per_w = B // NW
    CHUNK = 32
    HALF = CHUNK // 2
    n_chunks = b_per_w // CHUNK
    vecs_per_half = HALF * D // L

    mesh = plsc.VectorSubcoreMesh(core_axis_name="c", subcore_axis_name="s")

    @functools.partial(
        pl.kernel,
        mesh=mesh,
        out_type=jax.ShapeDtypeStruct((B, D), jnp.float32),
        scratch_types=[
            pltpu.VMEM((b_per_w,), jnp.int32),
        ]
        + [pltpu.VMEM((CHUNK, D), jnp.float32)] * NBUF
        + [pltpu.SemaphoreType.DMA] * (2 * NBUF + 1),
    )
    def k(idx_hbm, lut_hbm, out_hbm, idx_v, *rest):
        bufs = rest[:NBUF]
        gsems = rest[NBUF : 2 * NBUF]
        ssems = rest[2 * NBUF : 3 * NBUF]
        isem = rest[3 * NBUF]
        wid = lax.axis_index("s") * NC + lax.axis_index("c")
        base = wid * b_per_w

        n_pre = min(NBUF - 1, n_chunks)
        icopies = [
            pltpu.async_copy(
                idx_hbm.at[pl.ds(base + c * CHUNK, CHUNK)],
                idx_v.at[pl.ds(c * CHUNK, CHUNK)],
                isem,
            )
            for c in range(n_pre)
        ]
        rest_rows = b_per_w - n_pre * CHUNK
        irest = (
            pltpu.async_copy(
                idx_hbm.at[pl.ds(base + n_pre * CHUNK, rest_rows)],
                idx_v.at[pl.ds(n_pre * CHUNK, rest_rows)],
                isem,
            )
            if rest_rows
            else None
        )

        def gather(c):
            return pltpu.async_copy(
                lut_hbm.at[idx_v.at[pl.ds(c * CHUNK, CHUNK)]],
                bufs[c % NBUF],
                gsems[c % NBUF],
            )

        def scale_half(buf, h):
            def scale_body(i, carry):
                r = h * HALF + i // (D // L)
                j = i % (D // L)
                v = buf[r, pl.ds(j * L, L)]
                buf[r, pl.ds(j * L, L)] = v * jnp.float32(SCALE)
                return carry

            lax.fori_loop(0, vecs_per_half, scale_body, 0, unroll=8)

        gathers = {}
        for c in range(n_pre):
            icopies[c].wait()
            gathers[c] = gather(c)
        if irest is not None:
            irest.wait()

        stores = {}
        for c in range(n_chunks):
            nxt = c + NBUF - 1
            if nxt < n_chunks:
                if nxt - NBUF >= 0:
                    stores[nxt - NBUF][0].wait()
                    stores[nxt - NBUF][1].wait()
                gathers[nxt] = gather(nxt)
            gathers[c].wait()
            scale_half(bufs[c % NBUF], 0)
            s0 = pltpu.async_copy(
                bufs[c % NBUF].at[pl.ds(0, HALF)],
                out_hbm.at[pl.ds(base + c * CHUNK, HALF)],
                ssems[c % NBUF],
            )
            scale_half(bufs[c % NBUF], 1)
            s1 = pltpu.async_copy(
                bufs[c % NBUF].at[pl.ds(HALF, HALF)],
                out_hbm.at[pl.ds(base + c * CHUNK + HALF, HALF)],
                ssems[c % NBUF],
            )
            stores[c] = (s0, s1)
        for c in range(max(0, n_chunks - NBUF), n_chunks):
            stores[c][0].wait()
            stores[c][1].wait()

    return k


def kernel(input, lut):
    B = input.shape[0] * input.shape[1]
    idx = input.reshape((B,)).astype(jnp.int32)
    out = _make_sc_lookup(B, lut.shape[1])(idx, lut)
    return out.reshape(input.shape + (lut.shape[1],))


# speedup vs baseline: 1.0133x; 1.0133x over previous
import functools
from math import sqrt

import jax
import jax.numpy as jnp
from jax import lax
from jax.experimental import pallas as pl
from jax.experimental.pallas import tpu as pltpu
from jax.experimental.pallas import tpu_sc as plsc

D_MODEL = 1024
SCALE = sqrt(D_MODEL)


@functools.cache
def _make_sc_lookup(B: int, D: int):
    info = plsc.get_sparse_core_info()
    NC, NS, L = info.num_cores, info.num_subcores, info.num_lanes
    NW = NC * NS
    assert B % NW == 0 and D % L == 0
    b_per_w = B // NW
    CHUNK = 32
    n_chunks = b_per_w // CHUNK
    vecs_per_chunk = CHUNK * D // L

    mesh = plsc.VectorSubcoreMesh(core_axis_name="c", subcore_axis_name="s")

    @functools.partial(
        pl.kernel,
        mesh=mesh,
        out_type=jax.ShapeDtypeStruct((B, D), jnp.float32),
        scratch_types=[
            pltpu.VMEM((b_per_w,), jnp.int32),
            pltpu.VMEM((CHUNK, D), jnp.float32),
            pltpu.VMEM((CHUNK, D), jnp.float32),
            pltpu.SemaphoreType.DMA,
            pltpu.SemaphoreType.DMA,
            pltpu.SemaphoreType.DMA,
            pltpu.SemaphoreType.DMA,
        ],
    )
    def k(idx_hbm, lut_hbm, out_hbm, idx_v, rows0, rows1, g0, g1, s0, s1):
        wid = lax.axis_index("s") * NC + lax.axis_index("c")
        base = wid * b_per_w
        pltpu.sync_copy(idx_hbm.at[pl.ds(base, b_per_w)], idx_v)

        bufs = (rows0, rows1)
        gsems = (g0, g1)
        ssems = (s0, s1)

        def gather(c):
            return pltpu.async_copy(
                lut_hbm.at[idx_v.at[pl.ds(c * CHUNK, CHUNK)]],
                bufs[c % 2],
                gsems[c % 2],
            )

        def scale(buf):
            @plsc.parallel_loop(0, vecs_per_chunk, step=1, unroll=8)
            def scale_body(i):
                r = i // (D // L)
                j = i % (D // L)
                v = buf[r, pl.ds(j * L, L)]
                buf[r, pl.ds(j * L, L)] = v * jnp.float32(SCALE)

        gathers = {0: gather(0)}
        stores = {}
        for c in range(n_chunks):
            if c + 1 < n_chunks:
                if c - 1 >= 0:
                    stores[c - 1].wait()
                gathers[c + 1] = gather(c + 1)
            gathers[c].wait()
            scale(bufs[c % 2])
            stores[c] = pltpu.async_copy(
                bufs[c % 2],
                out_hbm.at[pl.ds(base + c * CHUNK, CHUNK)],
                ssems[c % 2],
            )
        stores[n_chunks - 2].wait()
        stores[n_chunks - 1].wait()

    return k


def kernel(input, lut):
    B = input.shape[0] * input.shape[1]
    idx = input.reshape((B,)).astype(jnp.int32)
    out = _make_sc_lookup(B, lut.shape[1])(idx, lut)
    return out.reshape(input.shape + (lut.shape[1],))


# speedup vs baseline: 1.0224x; 1.0089x over previous
import functools
from math import sqrt

import jax
import jax.numpy as jnp
from jax import lax
from jax.experimental import pallas as pl
from jax.experimental.pallas import tpu as pltpu
from jax.experimental.pallas import tpu_sc as plsc

D_MODEL = 1024
SCALE = sqrt(D_MODEL)


@functools.cache
def _make_sc_lookup(B: int, D: int):
    info = plsc.get_sparse_core_info()
    NC, NS, L = info.num_cores, info.num_subcores, info.num_lanes
    NW = NC * NS
    assert B % NW == 0 and D % L == 0
    b_per_w = B // NW
    CHUNK = 56
    full, last = divmod(b_per_w, CHUNK)
    sizes = [CHUNK] * full + ([last] if last else [])
    offs = [i * CHUNK for i in range(len(sizes))]
    n_chunks = len(sizes)

    mesh = plsc.VectorSubcoreMesh(core_axis_name="c", subcore_axis_name="s")

    @functools.partial(
        pl.kernel,
        mesh=mesh,
        out_type=jax.ShapeDtypeStruct((B, D), jnp.float32),
        scratch_types=[
            pltpu.VMEM((b_per_w,), jnp.int32),
            pltpu.VMEM((CHUNK, D), jnp.float32),
            pltpu.VMEM((CHUNK, D), jnp.float32),
            pltpu.SemaphoreType.DMA,
            pltpu.SemaphoreType.DMA,
            pltpu.SemaphoreType.DMA,
            pltpu.SemaphoreType.DMA,
        ],
    )
    def k(idx_hbm, lut_hbm, out_hbm, idx_v, rows0, rows1, g0, g1, s0, s1):
        wid = lax.axis_index("s") * NC + lax.axis_index("c")
        base = wid * b_per_w
        pltpu.sync_copy(idx_hbm.at[pl.ds(base, b_per_w)], idx_v)

        bufs = (rows0, rows1)
        gsems = (g0, g1)
        ssems = (s0, s1)

        def gather(c):
            dst = bufs[c % 2]
            if sizes[c] != CHUNK:
                dst = dst.at[pl.ds(0, sizes[c])]
            return pltpu.async_copy(
                lut_hbm.at[idx_v.at[pl.ds(offs[c], sizes[c])]],
                dst,
                gsems[c % 2],
            )

        def scale(buf, rows):
            @plsc.parallel_loop(0, rows * D // L, step=1, unroll=8)
            def scale_body(i):
                r = i // (D // L)
                j = i % (D // L)
                v = buf[r, pl.ds(j * L, L)]
                buf[r, pl.ds(j * L, L)] = v * jnp.float32(SCALE)

        gathers = {0: gather(0)}
        stores = {}
        for c in range(n_chunks):
            if c + 1 < n_chunks:
                if c - 1 >= 0:
                    stores[c - 1].wait()
                gathers[c + 1] = gather(c + 1)
            gathers[c].wait()
            scale(bufs[c % 2], sizes[c])
            src = bufs[c % 2]
            if sizes[c] != CHUNK:
                src = src.at[pl.ds(0, sizes[c])]
            stores[c] = pltpu.async_copy(
                src,
                out_hbm.at[pl.ds(base + offs[c], sizes[c])],
                ssems[c % 2],
            )
        stores[n_chunks - 2].wait()
        stores[n_chunks - 1].wait()

    return k


def kernel(input, lut):
    B = input.shape[0] * input.shape[1]
    idx = input.reshape((B,)).astype(jnp.int32)
    out = _make_sc_lookup(B, lut.shape[1])(idx, lut)
    return out.reshape(input.shape + (lut.shape[1],))
